# baseline (speedup 1.0000x reference)
import numpy as np
import jax
import jax.numpy as jnp
from jax.experimental import pallas as pl
from jax.experimental.pallas import tpu as pltpu

_IMG = 800.0
_THRESH = 0.25
_NMS_T = 0.5
_MAX_DET = 100
_CLIP = float(np.log(1000.0 / 16.0))
_B, _N, _D = 16, 4000, 1024
_H = 8


def _nms_iters(stash, s, keep, iota, n, unroll):
    x1 = stash[:, 0, :]
    y1 = stash[:, 1, :]
    x2 = stash[:, 2, :]
    y2 = stash[:, 3, :]
    area = stash[:, 5, :]

    def body(_, carry):
        s, keep = carry
        m = jnp.max(s, axis=1, keepdims=True)
        idx = jnp.argmax(s, axis=1, keepdims=True)
        one = iota == idx
        valid = jnp.where(m > 0.0, 1.0, 0.0)
        bx1 = jnp.sum(jnp.where(one, x1, 0.0), axis=1, keepdims=True)
        by1 = jnp.sum(jnp.where(one, y1, 0.0), axis=1, keepdims=True)
        bx2 = jnp.sum(jnp.where(one, x2, 0.0), axis=1, keepdims=True)
        by2 = jnp.sum(jnp.where(one, y2, 0.0), axis=1, keepdims=True)
        barea = jnp.sum(jnp.where(one, area, 0.0), axis=1, keepdims=True)
        ix1 = jnp.maximum(bx1, x1)
        iy1 = jnp.maximum(by1, y1)
        ix2 = jnp.minimum(bx2, x2)
        iy2 = jnp.minimum(by2, y2)
        inter = jnp.maximum(ix2 - ix1, 0.0) * jnp.maximum(iy2 - iy1, 0.0)
        supp = 3.0 * inter > area + (barea + 1e-9)
        s = jnp.where(one | supp, 0.0, s)
        keep = jnp.where(one, valid, keep)
        return s, keep

    return jax.lax.fori_loop(0, n, body, (s, keep), unroll=unroll)


def _fused_kernel(b_ref, w_ref, feats_ref, prop_ref, out_ref, fs_ref,
                  stash_ref, s1_ref, k1_ref):
    b = pl.program_id(0)
    feats = feats_ref[0]
    ot = jax.lax.dot_general(
        w_ref[...], feats, (((1,), (1,)), ((), ())),
        preferred_element_type=jnp.float32)

    l0 = ot[0:1, :] + b_ref[0]
    l1 = ot[1:2, :] + b_ref[1]
    mx = jnp.maximum(l0, l1)
    e0 = jnp.exp(l0 - mx)
    e1 = jnp.exp(l1 - mx)
    score = e1 / (e0 + e1)
    score = jnp.where(score > _THRESH, score, 0.0)

    x1 = prop_ref[0, 0:1, :]
    y1 = prop_ref[0, 1:2, :]
    x2 = prop_ref[0, 2:3, :]
    y2 = prop_ref[0, 3:4, :]
    wd = x2 - x1
    ht = y2 - y1
    cx = x1 + 0.5 * wd
    cy = y1 + 0.5 * ht
    dx = (ot[2:3, :] + b_ref[2]) / 10.0
    dy = (ot[3:4, :] + b_ref[3]) / 10.0
    dw = jnp.minimum((ot[4:5, :] + b_ref[4]) / 5.0, _CLIP)
    dh = jnp.minimum((ot[5:6, :] + b_ref[5]) / 5.0, _CLIP)
    pcx = dx * wd + cx
    pcy = dy * ht + cy
    pw = jnp.exp(dw) * wd
    ph = jnp.exp(dh) * ht
    bx1 = jnp.clip(pcx - 0.5 * pw, 0.0, _IMG)
    by1 = jnp.clip(pcy - 0.5 * ph, 0.0, _IMG)
    bx2 = jnp.clip(pcx + 0.5 * pw, 0.0, _IMG)
    by2 = jnp.clip(pcy + 0.5 * ph, 0.0, _IMG)
    area = jnp.maximum(bx2 - bx1, 0.0) * jnp.maximum(by2 - by1, 0.0)

    val = jnp.concatenate(
        [bx1, by1, bx2, by2, score, area,
         jnp.zeros((2, _N), jnp.float32)], axis=0)
    out_ref[0] = val
    stash_ref[pl.ds(b, 1)] = val[None]

    iota = jax.lax.broadcasted_iota(jnp.int32, (_H, _N), 1)

    @pl.when(b == _H)
    def _():
        s1_ref[...] = stash_ref[0:_H, 4, :]
        k1_ref[...] = jnp.zeros((_H, _N), jnp.float32)

    @pl.when(b >= _H)
    def _():
        s, keep = _nms_iters(stash_ref[0:_H], s1_ref[...], k1_ref[...],
                             iota, 12, 4)
        s1_ref[...] = s
        k1_ref[...] = keep

    @pl.when(b == _B - 1)
    def _():
        _, keep1 = _nms_iters(stash_ref[0:_H], s1_ref[...], k1_ref[...],
                              iota, _MAX_DET - 12 * _H, 4)
        fs_ref[0:_H, :] = stash_ref[0:_H, 4, :] * keep1
        s20 = stash_ref[_H:_B, 4, :]
        _, keep2 = _nms_iters(stash_ref[_H:_B], s20,
                              jnp.zeros((_H, _N), jnp.float32),
                              iota, _MAX_DET, 8)
        fs_ref[_H:_B, :] = s20 * keep2


def _forward(feats, proposals, W_cls, b_cls, W_box, b_box, interpret=False):
    w_all = jnp.concatenate([W_cls, W_box[4:8]], axis=0)
    w_pad = jnp.pad(w_all, ((0, 122), (0, 0)))
    b_all = jnp.concatenate(
        [b_cls, b_box[4:8], jnp.zeros((2,), jnp.float32)])
    prop_t = jnp.swapaxes(proposals, 1, 2)

    head, fs = pl.pallas_call(
        _fused_kernel,
        grid=(_B,),
        in_specs=[
            pl.BlockSpec(memory_space=pltpu.SMEM),
            pl.BlockSpec((128, _D), lambda i: (0, 0)),
            pl.BlockSpec((1, _N, _D), lambda i: (i, 0, 0)),
            pl.BlockSpec((1, 4, _N), lambda i: (i, 0, 0)),
        ],
        out_specs=[
            pl.BlockSpec((1, 8, _N), lambda i: (i, 0, 0)),
            pl.BlockSpec((_B, _N), lambda i: (0, 0)),
        ],
        out_shape=[
            jax.ShapeDtypeStruct((_B, 8, _N), jnp.float32),
            jax.ShapeDtypeStruct((_B, _N), jnp.float32),
        ],
        scratch_shapes=[
            pltpu.VMEM((_B, 8, _N), jnp.float32),
            pltpu.VMEM((_H, _N), jnp.float32),
            pltpu.VMEM((_H, _N), jnp.float32),
        ],
        compiler_params=pltpu.CompilerParams(
            dimension_semantics=("arbitrary",),
            vmem_limit_bytes=52 * 1024 * 1024,
        ),
        name="rcnn_fused",
        interpret=interpret,
    )(b_all, w_pad, feats, prop_t)

    out5 = jnp.concatenate([head[:, 0:4, :], fs[:, None, :]], axis=1)
    return jnp.swapaxes(out5, 1, 2)


def kernel(feats, proposals, W_cls, b_cls, W_box, b_box):
    return _forward(feats, proposals, W_cls, b_cls, W_box, b_box)


# speedup vs baseline: 1.2787x; 1.2787x over previous
import numpy as np
import jax
import jax.numpy as jnp
from jax.experimental import pallas as pl
from jax.experimental.pallas import tpu as pltpu

_IMG = 800.0
_THRESH = 0.25
_NMS_T = 0.5
_MAX_DET = 100
_CLIP = float(np.log(1000.0 / 16.0))
_B, _N, _D = 16, 4000, 1024


def _head_kernel(b_ref, w_ref, feats_ref, prop_ref, out_ref):
    feats = feats_ref[0]
    ot = jax.lax.dot_general(
        w_ref[...], feats, (((1,), (1,)), ((), ())),
        preferred_element_type=jnp.float32)

    l0 = ot[0:1, :] + b_ref[0]
    l1 = ot[1:2, :] + b_ref[1]
    m = jnp.maximum(l0, l1)
    e0 = jnp.exp(l0 - m)
    e1 = jnp.exp(l1 - m)
    score = e1 / (e0 + e1)
    score = jnp.where(score > _THRESH, score, 0.0)

    x1 = prop_ref[0, 0:1, :]
    y1 = prop_ref[0, 1:2, :]
    x2 = prop_ref[0, 2:3, :]
    y2 = prop_ref[0, 3:4, :]
    wd = x2 - x1
    ht = y2 - y1
    cx = x1 + 0.5 * wd
    cy = y1 + 0.5 * ht
    dx = (ot[2:3, :] + b_ref[2]) / 10.0
    dy = (ot[3:4, :] + b_ref[3]) / 10.0
    dw = jnp.minimum((ot[4:5, :] + b_ref[4]) / 5.0, _CLIP)
    dh = jnp.minimum((ot[5:6, :] + b_ref[5]) / 5.0, _CLIP)
    pcx = dx * wd + cx
    pcy = dy * ht + cy
    pw = jnp.exp(dw) * wd
    ph = jnp.exp(dh) * ht
    bx1 = jnp.clip(pcx - 0.5 * pw, 0.0, _IMG)
    by1 = jnp.clip(pcy - 0.5 * ph, 0.0, _IMG)
    bx2 = jnp.clip(pcx + 0.5 * pw, 0.0, _IMG)
    by2 = jnp.clip(pcy + 0.5 * ph, 0.0, _IMG)
    area = jnp.maximum(bx2 - bx1, 0.0) * jnp.maximum(by2 - by1, 0.0)

    out_ref[0, 0:1, :] = bx1
    out_ref[0, 1:2, :] = by1
    out_ref[0, 2:3, :] = bx2
    out_ref[0, 3:4, :] = by2
    out_ref[0, 4:5, :] = score
    out_ref[0, 5:6, :] = area
    out_ref[0, 6:8, :] = jnp.zeros((2, _N), jnp.float32)


def _nms_kernel(head_ref, out_ref):
    x1 = head_ref[:, 0, :]
    y1 = head_ref[:, 1, :]
    x2 = head_ref[:, 2, :]
    y2 = head_ref[:, 3, :]
    s0 = head_ref[:, 4, :]
    area = head_ref[:, 5, :]
    iota = jax.lax.broadcasted_iota(jnp.int32, (_B, _N), 1)

    def body(_, carry):
        s, keep = carry
        idx = jnp.argmax(s, axis=1, keepdims=True)
        one = iota == idx
        m = jnp.sum(jnp.where(one, s, 0.0), axis=1, keepdims=True)
        valid = jnp.where(m > 0.0, 1.0, 0.0)
        bx1 = jnp.sum(jnp.where(one, x1, 0.0), axis=1, keepdims=True)
        by1 = jnp.sum(jnp.where(one, y1, 0.0), axis=1, keepdims=True)
        bx2 = jnp.sum(jnp.where(one, x2, 0.0), axis=1, keepdims=True)
        by2 = jnp.sum(jnp.where(one, y2, 0.0), axis=1, keepdims=True)
        barea = jnp.sum(jnp.where(one, area, 0.0), axis=1, keepdims=True)
        ix1 = jnp.maximum(bx1, x1)
        iy1 = jnp.maximum(by1, y1)
        ix2 = jnp.minimum(bx2, x2)
        iy2 = jnp.minimum(by2, y2)
        inter = jnp.maximum(ix2 - ix1, 0.0) * jnp.maximum(iy2 - iy1, 0.0)
        supp = 3.0 * inter > area + (barea + 1e-9)
        s = jnp.where(one | supp, 0.0, s)
        keep = jnp.where(one, valid, keep)
        return s, keep

    _, keep = jax.lax.fori_loop(
        0, _MAX_DET, body, (s0, jnp.zeros((_B, _N), jnp.float32)),
        unroll=8)
    out_ref[...] = s0 * keep


def _forward(feats, proposals, W_cls, b_cls, W_box, b_box, interpret=False):
    w_all = jnp.concatenate([W_cls, W_box[4:8]], axis=0)
    w_pad = jnp.pad(w_all, ((0, 122), (0, 0)))
    b_all = jnp.concatenate(
        [b_cls, b_box[4:8], jnp.zeros((2,), jnp.float32)])
    prop_t = jnp.swapaxes(proposals, 1, 2)

    head = pl.pallas_call(
        _head_kernel,
        grid=(_B,),
        in_specs=[
            pl.BlockSpec(memory_space=pltpu.SMEM),
            pl.BlockSpec((128, _D), lambda i: (0, 0)),
            pl.BlockSpec((1, _N, _D), lambda i: (i, 0, 0)),
            pl.BlockSpec((1, 4, _N), lambda i: (i, 0, 0)),
        ],
        out_specs=pl.BlockSpec((1, 8, _N), lambda i: (i, 0, 0)),
        out_shape=jax.ShapeDtypeStruct((_B, 8, _N), jnp.float32),
        compiler_params=pltpu.CompilerParams(
            dimension_semantics=("arbitrary",),
            vmem_limit_bytes=52 * 1024 * 1024,
        ),
        name="rcnn_head",
        interpret=interpret,
    )(b_all, w_pad, feats, prop_t)

    fs = pl.pallas_call(
        _nms_kernel,
        in_specs=[pl.BlockSpec((_B, 8, _N), lambda: (0, 0, 0))],
        out_specs=pl.BlockSpec((_B, _N), lambda: (0, 0)),
        out_shape=jax.ShapeDtypeStruct((_B, _N), jnp.float32),
        name="rcnn_nms",
        interpret=interpret,
    )(head)

    out5 = jnp.concatenate([head[:, 0:4, :], fs[:, None, :]], axis=1)
    return jnp.swapaxes(out5, 1, 2)


def kernel(feats, proposals, W_cls, b_cls, W_box, b_box):
    return _forward(feats, proposals, W_cls, b_cls, W_box, b_box)


# speedup vs baseline: 1.2873x; 1.0068x over previous
import numpy as np
import jax
import jax.numpy as jnp
from jax.experimental import pallas as pl
from jax.experimental.pallas import tpu as pltpu

_IMG = 800.0
_THRESH = 0.25
_NMS_T = 0.5
_MAX_DET = 100
_CLIP = float(np.log(1000.0 / 16.0))
_B, _N, _D = 16, 4000, 1024


def _head_kernel(b_ref, w_ref, feats_ref, prop_ref, out_ref):
    feats = feats_ref[0]
    ot = jax.lax.dot_general(
        w_ref[...], feats, (((1,), (1,)), ((), ())),
        preferred_element_type=jnp.float32)

    l0 = ot[0:1, :] + b_ref[0]
    l1 = ot[1:2, :] + b_ref[1]
    m = jnp.maximum(l0, l1)
    e0 = jnp.exp(l0 - m)
    e1 = jnp.exp(l1 - m)
    score = e1 / (e0 + e1)
    score = jnp.where(score > _THRESH, score, 0.0)

    x1 = prop_ref[0, 0:1, :]
    y1 = prop_ref[0, 1:2, :]
    x2 = prop_ref[0, 2:3, :]
    y2 = prop_ref[0, 3:4, :]
    wd = x2 - x1
    ht = y2 - y1
    cx = x1 + 0.5 * wd
    cy = y1 + 0.5 * ht
    dx = (ot[2:3, :] + b_ref[2]) / 10.0
    dy = (ot[3:4, :] + b_ref[3]) / 10.0
    dw = jnp.minimum((ot[4:5, :] + b_ref[4]) / 5.0, _CLIP)
    dh = jnp.minimum((ot[5:6, :] + b_ref[5]) / 5.0, _CLIP)
    pcx = dx * wd + cx
    pcy = dy * ht + cy
    pw = jnp.exp(dw) * wd
    ph = jnp.exp(dh) * ht
    bx1 = jnp.clip(pcx - 0.5 * pw, 0.0, _IMG)
    by1 = jnp.clip(pcy - 0.5 * ph, 0.0, _IMG)
    bx2 = jnp.clip(pcx + 0.5 * pw, 0.0, _IMG)
    by2 = jnp.clip(pcy + 0.5 * ph, 0.0, _IMG)
    area = jnp.maximum(bx2 - bx1, 0.0) * jnp.maximum(by2 - by1, 0.0)

    out_ref[0, 0:1, :] = bx1
    out_ref[0, 1:2, :] = by1
    out_ref[0, 2:3, :] = bx2
    out_ref[0, 3:4, :] = by2
    out_ref[0, 4:5, :] = score
    out_ref[0, 5:6, :] = area
    out_ref[0, 6:8, :] = jnp.zeros((2, _N), jnp.float32)


_H = 8


def _nms_kernel(head_ref, out_ref):
    iota = jax.lax.broadcasted_iota(jnp.int32, (_H, _N), 1)

    def make_half(lo, hi):
        x1 = head_ref[lo:hi, 0, :]
        y1 = head_ref[lo:hi, 1, :]
        x2 = head_ref[lo:hi, 2, :]
        y2 = head_ref[lo:hi, 3, :]
        s0 = head_ref[lo:hi, 4, :]
        area = head_ref[lo:hi, 5, :]

        def step(s, keep):
            m = jnp.max(s, axis=1, keepdims=True)
            idx = jnp.argmax(s, axis=1, keepdims=True)
            one = iota == idx
            valid = jnp.where(m > 0.0, 1.0, 0.0)
            bx1 = jnp.sum(jnp.where(one, x1, 0.0), axis=1, keepdims=True)
            by1 = jnp.sum(jnp.where(one, y1, 0.0), axis=1, keepdims=True)
            bx2 = jnp.sum(jnp.where(one, x2, 0.0), axis=1, keepdims=True)
            by2 = jnp.sum(jnp.where(one, y2, 0.0), axis=1, keepdims=True)
            barea = jnp.sum(jnp.where(one, area, 0.0), axis=1,
                            keepdims=True)
            ix1 = jnp.maximum(bx1, x1)
            iy1 = jnp.maximum(by1, y1)
            ix2 = jnp.minimum(bx2, x2)
            iy2 = jnp.minimum(by2, y2)
            inter = jnp.maximum(ix2 - ix1, 0.0) * \
                jnp.maximum(iy2 - iy1, 0.0)
            supp = 3.0 * inter > area + (barea + 1e-9)
            s = jnp.where(one | supp, 0.0, s)
            keep = jnp.where(one, valid, keep)
            return s, keep

        return s0, step

    s0a, step_a = make_half(0, _H)
    s0b, step_b = make_half(_H, _B)
    zeros = jnp.zeros((_H, _N), jnp.float32)

    def body(_, carry):
        sa, ka, sb, kb = carry
        sa, ka = step_a(sa, ka)
        sb, kb = step_b(sb, kb)
        return sa, ka, sb, kb

    _, ka, _, kb = jax.lax.fori_loop(
        0, _MAX_DET, body, (s0a, zeros, s0b, zeros), unroll=4)
    out_ref[0:_H, :] = s0a * ka
    out_ref[_H:_B, :] = s0b * kb


def _forward(feats, proposals, W_cls, b_cls, W_box, b_box, interpret=False):
    w_all = jnp.concatenate([W_cls, W_box[4:8]], axis=0)
    w_pad = jnp.pad(w_all, ((0, 122), (0, 0)))
    b_all = jnp.concatenate(
        [b_cls, b_box[4:8], jnp.zeros((2,), jnp.float32)])
    prop_t = jnp.swapaxes(proposals, 1, 2)

    head = pl.pallas_call(
        _head_kernel,
        grid=(_B,),
        in_specs=[
            pl.BlockSpec(memory_space=pltpu.SMEM),
            pl.BlockSpec((128, _D), lambda i: (0, 0)),
            pl.BlockSpec((1, _N, _D), lambda i: (i, 0, 0)),
            pl.BlockSpec((1, 4, _N), lambda i: (i, 0, 0)),
        ],
        out_specs=pl.BlockSpec((1, 8, _N), lambda i: (i, 0, 0)),
        out_shape=jax.ShapeDtypeStruct((_B, 8, _N), jnp.float32),
        compiler_params=pltpu.CompilerParams(
            dimension_semantics=("arbitrary",),
            vmem_limit_bytes=52 * 1024 * 1024,
        ),
        name="rcnn_head",
        interpret=interpret,
    )(b_all, w_pad, feats, prop_t)

    fs = pl.pallas_call(
        _nms_kernel,
        in_specs=[pl.BlockSpec((_B, 8, _N), lambda: (0, 0, 0))],
        out_specs=pl.BlockSpec((_B, _N), lambda: (0, 0)),
        out_shape=jax.ShapeDtypeStruct((_B, _N), jnp.float32),
        name="rcnn_nms",
        interpret=interpret,
    )(head)

    out5 = jnp.concatenate([head[:, 0:4, :], fs[:, None, :]], axis=1)
    return jnp.swapaxes(out5, 1, 2)


def kernel(feats, proposals, W_cls, b_cls, W_box, b_box):
    return _forward(feats, proposals, W_cls, b_cls, W_box, b_box)


# speedup vs baseline: 1.3174x; 1.0233x over previous
import numpy as np
import jax
import jax.numpy as jnp
from jax.experimental import pallas as pl
from jax.experimental.pallas import tpu as pltpu

_IMG = 800.0
_THRESH = 0.25
_NMS_T = 0.5
_MAX_DET = 100
_CLIP = float(np.log(1000.0 / 16.0))
_B, _N, _D = 16, 4000, 1024


def _head_kernel(b_ref, w_ref, feats_ref, prop_ref, out_ref):
    feats = feats_ref[0]
    ot = jax.lax.dot_general(
        w_ref[...], feats, (((1,), (1,)), ((), ())),
        preferred_element_type=jnp.float32)

    l0 = ot[0:1, :] + b_ref[0]
    l1 = ot[1:2, :] + b_ref[1]
    m = jnp.maximum(l0, l1)
    e0 = jnp.exp(l0 - m)
    e1 = jnp.exp(l1 - m)
    score = e1 / (e0 + e1)
    score = jnp.where(score > _THRESH, score, 0.0)

    x1 = prop_ref[0, 0:1, :]
    y1 = prop_ref[0, 1:2, :]
    x2 = prop_ref[0, 2:3, :]
    y2 = prop_ref[0, 3:4, :]
    wd = x2 - x1
    ht = y2 - y1
    cx = x1 + 0.5 * wd
    cy = y1 + 0.5 * ht
    dx = (ot[2:3, :] + b_ref[2]) / 10.0
    dy = (ot[3:4, :] + b_ref[3]) / 10.0
    dw = jnp.minimum((ot[4:5, :] + b_ref[4]) / 5.0, _CLIP)
    dh = jnp.minimum((ot[5:6, :] + b_ref[5]) / 5.0, _CLIP)
    pcx = dx * wd + cx
    pcy = dy * ht + cy
    pw = jnp.exp(dw) * wd
    ph = jnp.exp(dh) * ht
    bx1 = jnp.clip(pcx - 0.5 * pw, 0.0, _IMG)
    by1 = jnp.clip(pcy - 0.5 * ph, 0.0, _IMG)
    bx2 = jnp.clip(pcx + 0.5 * pw, 0.0, _IMG)
    by2 = jnp.clip(pcy + 0.5 * ph, 0.0, _IMG)
    area = jnp.maximum(bx2 - bx1, 0.0) * jnp.maximum(by2 - by1, 0.0)

    out_ref[0, 0:1, :] = bx1
    out_ref[0, 1:2, :] = by1
    out_ref[0, 2:3, :] = bx2
    out_ref[0, 3:4, :] = by2
    out_ref[0, 4:5, :] = score
    out_ref[0, 5:6, :] = area
    out_ref[0, 6:8, :] = jnp.zeros((2, _N), jnp.float32)


_H = 8


def _nms_kernel(head_ref, out_ref):
    iota = jax.lax.broadcasted_iota(jnp.int32, (_H, _N), 1)

    def make_half(lo, hi):
        x1 = head_ref[lo:hi, 0, :]
        y1 = head_ref[lo:hi, 1, :]
        x2 = head_ref[lo:hi, 2, :]
        y2 = head_ref[lo:hi, 3, :]
        s0 = head_ref[lo:hi, 4, :]
        area = head_ref[lo:hi, 5, :]

        def step(s, keep):
            m = jnp.max(s, axis=1, keepdims=True)
            idx = jnp.argmax(s, axis=1, keepdims=True)
            one = iota == idx
            valid = jnp.where(m > 0.0, 1.0, 0.0)
            bx1 = jnp.sum(jnp.where(one, x1, 0.0), axis=1, keepdims=True)
            by1 = jnp.sum(jnp.where(one, y1, 0.0), axis=1, keepdims=True)
            bx2 = jnp.sum(jnp.where(one, x2, 0.0), axis=1, keepdims=True)
            by2 = jnp.sum(jnp.where(one, y2, 0.0), axis=1, keepdims=True)
            barea = jnp.sum(jnp.where(one, area, 0.0), axis=1,
                            keepdims=True)
            ix1 = jnp.maximum(bx1, x1)
            iy1 = jnp.maximum(by1, y1)
            ix2 = jnp.minimum(bx2, x2)
            iy2 = jnp.minimum(by2, y2)
            inter = jnp.maximum(ix2 - ix1, 0.0) * \
                jnp.maximum(iy2 - iy1, 0.0)
            supp = 3.0 * inter > area + (barea + 1e-9)
            s = jnp.where(one | supp, 0.0, s)
            keep = jnp.where(one, valid, keep)
            return s, keep

        return s0, step

    s0a, step_a = make_half(0, _H)
    s0b, step_b = make_half(_H, _B)
    zeros = jnp.zeros((_H, _N), jnp.float32)

    def body(_, carry):
        sa, ka, sb, kb = carry
        sa, ka = step_a(sa, ka)
        sb, kb = step_b(sb, kb)
        return sa, ka, sb, kb

    _, ka, _, kb = jax.lax.fori_loop(
        0, _MAX_DET, body, (s0a, zeros, s0b, zeros), unroll=8)
    out_ref[0:_H, :] = s0a * ka
    out_ref[_H:_B, :] = s0b * kb


def _forward(feats, proposals, W_cls, b_cls, W_box, b_box, interpret=False):
    w_all = jnp.concatenate([W_cls, W_box[4:8]], axis=0)
    w_pad = jnp.pad(w_all, ((0, 122), (0, 0)))
    b_all = jnp.concatenate(
        [b_cls, b_box[4:8], jnp.zeros((2,), jnp.float32)])
    prop_t = jnp.swapaxes(proposals, 1, 2)

    head = pl.pallas_call(
        _head_kernel,
        grid=(_B,),
        in_specs=[
            pl.BlockSpec(memory_space=pltpu.SMEM),
            pl.BlockSpec((128, _D), lambda i: (0, 0)),
            pl.BlockSpec((1, _N, _D), lambda i: (i, 0, 0)),
            pl.BlockSpec((1, 4, _N), lambda i: (i, 0, 0)),
        ],
        out_specs=pl.BlockSpec((1, 8, _N), lambda i: (i, 0, 0)),
        out_shape=jax.ShapeDtypeStruct((_B, 8, _N), jnp.float32),
        compiler_params=pltpu.CompilerParams(
            dimension_semantics=("arbitrary",),
            vmem_limit_bytes=52 * 1024 * 1024,
        ),
        name="rcnn_head",
        interpret=interpret,
    )(b_all, w_pad, feats, prop_t)

    fs = pl.pallas_call(
        _nms_kernel,
        in_specs=[pl.BlockSpec((_B, 8, _N), lambda: (0, 0, 0))],
        out_specs=pl.BlockSpec((_B, _N), lambda: (0, 0)),
        out_shape=jax.ShapeDtypeStruct((_B, _N), jnp.float32),
        name="rcnn_nms",
        interpret=interpret,
    )(head)

    out5 = jnp.concatenate([head[:, 0:4, :], fs[:, None, :]], axis=1)
    return jnp.swapaxes(out5, 1, 2)


def kernel(feats, proposals, W_cls, b_cls, W_box, b_box):
    return _forward(feats, proposals, W_cls, b_cls, W_box, b_box)


# speedup vs baseline: 1.3322x; 1.0113x over previous
import numpy as np
import jax
import jax.numpy as jnp
from jax.experimental import pallas as pl
from jax.experimental.pallas import tpu as pltpu

_IMG = 800.0
_THRESH = 0.25
_NMS_T = 0.5
_MAX_DET = 100
_CLIP = float(np.log(1000.0 / 16.0))
_B, _N, _D = 16, 4000, 1024
_H = 8


def _run_nms(stash_ref, fs_ref):
    iota = jax.lax.broadcasted_iota(jnp.int32, (_H, _N), 1)

    def make_half(lo, hi):
        x1 = stash_ref[lo:hi, 0, :]
        y1 = stash_ref[lo:hi, 1, :]
        x2 = stash_ref[lo:hi, 2, :]
        y2 = stash_ref[lo:hi, 3, :]
        s0 = stash_ref[lo:hi, 4, :]
        area = stash_ref[lo:hi, 5, :]

        def step(s, keep):
            m = jnp.max(s, axis=1, keepdims=True)
            idx = jnp.argmax(s, axis=1, keepdims=True)
            one = iota == idx
            valid = jnp.where(m > 0.0, 1.0, 0.0)
            bx1 = jnp.sum(jnp.where(one, x1, 0.0), axis=1, keepdims=True)
            by1 = jnp.sum(jnp.where(one, y1, 0.0), axis=1, keepdims=True)
            bx2 = jnp.sum(jnp.where(one, x2, 0.0), axis=1, keepdims=True)
            by2 = jnp.sum(jnp.where(one, y2, 0.0), axis=1, keepdims=True)
            barea = jnp.sum(jnp.where(one, area, 0.0), axis=1,
                            keepdims=True)
            ix1 = jnp.maximum(bx1, x1)
            iy1 = jnp.maximum(by1, y1)
            ix2 = jnp.minimum(bx2, x2)
            iy2 = jnp.minimum(by2, y2)
            inter = jnp.maximum(ix2 - ix1, 0.0) * \
                jnp.maximum(iy2 - iy1, 0.0)
            supp = 3.0 * inter > area + (barea + 1e-9)
            s = jnp.where(one | supp, 0.0, s)
            keep = jnp.where(one, valid, keep)
            return s, keep

        return s0, step

    s0a, step_a = make_half(0, _H)
    s0b, step_b = make_half(_H, _B)
    zeros = jnp.zeros((_H, _N), jnp.float32)

    def body(_, carry):
        sa, ka, sb, kb = carry
        sa, ka = step_a(sa, ka)
        sb, kb = step_b(sb, kb)
        return sa, ka, sb, kb

    _, ka, _, kb = jax.lax.fori_loop(
        0, _MAX_DET, body, (s0a, zeros, s0b, zeros), unroll=8)
    fs_ref[0:_H, :] = s0a * ka
    fs_ref[_H:_B, :] = s0b * kb


def _fused_kernel(b_ref, w_ref, feats_ref, prop_ref, out_ref, fs_ref,
                  stash_ref):
    b = pl.program_id(0)
    feats = feats_ref[0]
    ot = jax.lax.dot_general(
        w_ref[...], feats, (((1,), (1,)), ((), ())),
        preferred_element_type=jnp.float32)

    l0 = ot[0:1, :] + b_ref[0]
    l1 = ot[1:2, :] + b_ref[1]
    mx = jnp.maximum(l0, l1)
    e0 = jnp.exp(l0 - mx)
    e1 = jnp.exp(l1 - mx)
    score = e1 / (e0 + e1)
    score = jnp.where(score > _THRESH, score, 0.0)

    x1 = prop_ref[0, 0:1, :]
    y1 = prop_ref[0, 1:2, :]
    x2 = prop_ref[0, 2:3, :]
    y2 = prop_ref[0, 3:4, :]
    wd = x2 - x1
    ht = y2 - y1
    cx = x1 + 0.5 * wd
    cy = y1 + 0.5 * ht
    dx = (ot[2:3, :] + b_ref[2]) / 10.0
    dy = (ot[3:4, :] + b_ref[3]) / 10.0
    dw = jnp.minimum((ot[4:5, :] + b_ref[4]) / 5.0, _CLIP)
    dh = jnp.minimum((ot[5:6, :] + b_ref[5]) / 5.0, _CLIP)
    pcx = dx * wd + cx
    pcy = dy * ht + cy
    pw = jnp.exp(dw) * wd
    ph = jnp.exp(dh) * ht
    bx1 = jnp.clip(pcx - 0.5 * pw, 0.0, _IMG)
    by1 = jnp.clip(pcy - 0.5 * ph, 0.0, _IMG)
    bx2 = jnp.clip(pcx + 0.5 * pw, 0.0, _IMG)
    by2 = jnp.clip(pcy + 0.5 * ph, 0.0, _IMG)
    area = jnp.maximum(bx2 - bx1, 0.0) * jnp.maximum(by2 - by1, 0.0)

    val = jnp.concatenate(
        [bx1, by1, bx2, by2, score, area,
         jnp.zeros((2, _N), jnp.float32)], axis=0)
    out_ref[0] = val
    stash_ref[pl.ds(b, 1)] = val[None]

    @pl.when(b == _B - 1)
    def _():
        _run_nms(stash_ref, fs_ref)


def _forward(feats, proposals, W_cls, b_cls, W_box, b_box, interpret=False):
    w_all = jnp.concatenate([W_cls, W_box[4:8]], axis=0)
    w_pad = jnp.pad(w_all, ((0, 122), (0, 0)))
    b_all = jnp.concatenate(
        [b_cls, b_box[4:8], jnp.zeros((2,), jnp.float32)])
    prop_t = jnp.swapaxes(proposals, 1, 2)

    head, fs = pl.pallas_call(
        _fused_kernel,
        grid=(_B,),
        in_specs=[
            pl.BlockSpec(memory_space=pltpu.SMEM),
            pl.BlockSpec((128, _D), lambda i: (0, 0)),
            pl.BlockSpec((1, _N, _D), lambda i: (i, 0, 0)),
            pl.BlockSpec((1, 4, _N), lambda i: (i, 0, 0)),
        ],
        out_specs=[
            pl.BlockSpec((1, 8, _N), lambda i: (i, 0, 0)),
            pl.BlockSpec((_B, _N), lambda i: (0, 0)),
        ],
        out_shape=[
            jax.ShapeDtypeStruct((_B, 8, _N), jnp.float32),
            jax.ShapeDtypeStruct((_B, _N), jnp.float32),
        ],
        scratch_shapes=[
            pltpu.VMEM((_B, 8, _N), jnp.float32),
        ],
        compiler_params=pltpu.CompilerParams(
            dimension_semantics=("arbitrary",),
            vmem_limit_bytes=52 * 1024 * 1024,
        ),
        name="rcnn_fused",
        interpret=interpret,
    )(b_all, w_pad, feats, prop_t)

    out5 = jnp.concatenate([head[:, 0:4, :], fs[:, None, :]], axis=1)
    return jnp.swapaxes(out5, 1, 2)


def kernel(feats, proposals, W_cls, b_cls, W_box, b_box):
    return _forward(feats, proposals, W_cls, b_cls, W_box, b_box)


# speedup vs baseline: 1.3572x; 1.0187x over previous
import numpy as np
import jax
import jax.numpy as jnp
from jax.experimental import pallas as pl
from jax.experimental.pallas import tpu as pltpu

_IMG = 800.0
_THRESH = 0.25
_NMS_T = 0.5
_MAX_DET = 100
_CLIP = float(np.log(1000.0 / 16.0))
_B, _N, _D = 16, 4000, 1024
_H = 8


def _run_nms(stash_ref, fs_ref):
    iota = jax.lax.broadcasted_iota(jnp.int32, (_H, _N), 1)

    def make_half(lo, hi):
        x1 = stash_ref[lo:hi, 0, :]
        y1 = stash_ref[lo:hi, 1, :]
        x2 = stash_ref[lo:hi, 2, :]
        y2 = stash_ref[lo:hi, 3, :]
        s0 = stash_ref[lo:hi, 4, :]
        area = stash_ref[lo:hi, 5, :]

        def step(s, keep):
            m = jnp.max(s, axis=1, keepdims=True)
            idx = jnp.argmax(s, axis=1, keepdims=True)
            one = iota == idx
            valid = jnp.where(m > 0.0, 1.0, 0.0)
            bx1 = jnp.sum(jnp.where(one, x1, 0.0), axis=1, keepdims=True)
            by1 = jnp.sum(jnp.where(one, y1, 0.0), axis=1, keepdims=True)
            bx2 = jnp.sum(jnp.where(one, x2, 0.0), axis=1, keepdims=True)
            by2 = jnp.sum(jnp.where(one, y2, 0.0), axis=1, keepdims=True)
            barea = jnp.maximum(bx2 - bx1, 0.0) * \
                jnp.maximum(by2 - by1, 0.0)
            ix1 = jnp.maximum(bx1, x1)
            iy1 = jnp.maximum(by1, y1)
            ix2 = jnp.minimum(bx2, x2)
            iy2 = jnp.minimum(by2, y2)
            inter = jnp.maximum(ix2 - ix1, 0.0) * \
                jnp.maximum(iy2 - iy1, 0.0)
            supp = 3.0 * inter > area + (barea + 1e-9)
            s = jnp.where(one | supp, 0.0, s)
            keep = jnp.where(one, valid, keep)
            return s, keep

        return s0, step

    s0a, step_a = make_half(0, _H)
    s0b, step_b = make_half(_H, _B)
    zeros = jnp.zeros((_H, _N), jnp.float32)

    def body(_, carry):
        sa, ka, sb, kb = carry
        sa, ka = step_a(sa, ka)
        sb, kb = step_b(sb, kb)
        return sa, ka, sb, kb

    _, ka, _, kb = jax.lax.fori_loop(
        0, _MAX_DET, body, (s0a, zeros, s0b, zeros), unroll=8)
    fs_ref[0:_H, :] = s0a * ka
    fs_ref[_H:_B, :] = s0b * kb


def _fused_kernel(b_ref, w_ref, feats_ref, prop_ref, out_ref, fs_ref,
                  stash_ref):
    b = pl.program_id(0)
    feats = feats_ref[0]
    ot = jax.lax.dot_general(
        w_ref[...], feats, (((1,), (1,)), ((), ())),
        preferred_element_type=jnp.float32)

    l0 = ot[0:1, :] + b_ref[0]
    l1 = ot[1:2, :] + b_ref[1]
    mx = jnp.maximum(l0, l1)
    e0 = jnp.exp(l0 - mx)
    e1 = jnp.exp(l1 - mx)
    score = e1 / (e0 + e1)
    score = jnp.where(score > _THRESH, score, 0.0)

    x1 = prop_ref[0, 0:1, :]
    y1 = prop_ref[0, 1:2, :]
    x2 = prop_ref[0, 2:3, :]
    y2 = prop_ref[0, 3:4, :]
    wd = x2 - x1
    ht = y2 - y1
    cx = x1 + 0.5 * wd
    cy = y1 + 0.5 * ht
    dx = (ot[2:3, :] + b_ref[2]) / 10.0
    dy = (ot[3:4, :] + b_ref[3]) / 10.0
    dw = jnp.minimum((ot[4:5, :] + b_ref[4]) / 5.0, _CLIP)
    dh = jnp.minimum((ot[5:6, :] + b_ref[5]) / 5.0, _CLIP)
    pcx = dx * wd + cx
    pcy = dy * ht + cy
    pw = jnp.exp(dw) * wd
    ph = jnp.exp(dh) * ht
    bx1 = jnp.clip(pcx - 0.5 * pw, 0.0, _IMG)
    by1 = jnp.clip(pcy - 0.5 * ph, 0.0, _IMG)
    bx2 = jnp.clip(pcx + 0.5 * pw, 0.0, _IMG)
    by2 = jnp.clip(pcy + 0.5 * ph, 0.0, _IMG)
    area = jnp.maximum(bx2 - bx1, 0.0) * jnp.maximum(by2 - by1, 0.0)

    val = jnp.concatenate(
        [bx1, by1, bx2, by2, score, area,
         jnp.zeros((2, _N), jnp.float32)], axis=0)
    out_ref[0] = val
    stash_ref[pl.ds(b, 1)] = val[None]

    @pl.when(b == _B - 1)
    def _():
        _run_nms(stash_ref, fs_ref)


def _forward(feats, proposals, W_cls, b_cls, W_box, b_box, interpret=False):
    w_all = jnp.concatenate([W_cls, W_box[4:8]], axis=0)
    w_pad = jnp.pad(w_all, ((0, 122), (0, 0)))
    b_all = jnp.concatenate(
        [b_cls, b_box[4:8], jnp.zeros((2,), jnp.float32)])
    prop_t = jnp.swapaxes(proposals, 1, 2)

    head, fs = pl.pallas_call(
        _fused_kernel,
        grid=(_B,),
        in_specs=[
            pl.BlockSpec(memory_space=pltpu.SMEM),
            pl.BlockSpec((128, _D), lambda i: (0, 0)),
            pl.BlockSpec((1, _N, _D), lambda i: (i, 0, 0)),
            pl.BlockSpec((1, 4, _N), lambda i: (i, 0, 0)),
        ],
        out_specs=[
            pl.BlockSpec((1, 8, _N), lambda i: (i, 0, 0)),
            pl.BlockSpec((_B, _N), lambda i: (0, 0)),
        ],
        out_shape=[
            jax.ShapeDtypeStruct((_B, 8, _N), jnp.float32),
            jax.ShapeDtypeStruct((_B, _N), jnp.float32),
        ],
        scratch_shapes=[
            pltpu.VMEM((_B, 8, _N), jnp.float32),
        ],
        compiler_params=pltpu.CompilerParams(
            dimension_semantics=("arbitrary",),
            vmem_limit_bytes=52 * 1024 * 1024,
        ),
        name="rcnn_fused",
        interpret=interpret,
    )(b_all, w_pad, feats, prop_t)

    out5 = jnp.concatenate([head[:, 0:4, :], fs[:, None, :]], axis=1)
    return jnp.swapaxes(out5, 1, 2)


def kernel(feats, proposals, W_cls, b_cls, W_box, b_box):
    return _forward(feats, proposals, W_cls, b_cls, W_box, b_box)


# speedup vs baseline: 1.3655x; 1.0061x over previous
import numpy as np
import jax
import jax.numpy as jnp
from jax.experimental import pallas as pl
from jax.experimental.pallas import tpu as pltpu

_IMG = 800.0
_THRESH = 0.25
_NMS_T = 0.5
_MAX_DET = 100
_CLIP = float(np.log(1000.0 / 16.0))
_B, _N, _D = 16, 4000, 1024
_H = 8


def _run_nms(stash_ref, fs_ref):
    iota = jax.lax.broadcasted_iota(jnp.int32, (_H, _N), 1)

    def make_half(lo, hi):
        x1 = stash_ref[lo:hi, 0, :]
        y1 = stash_ref[lo:hi, 1, :]
        x2 = stash_ref[lo:hi, 2, :]
        y2 = stash_ref[lo:hi, 3, :]
        s0 = stash_ref[lo:hi, 4, :]
        area = stash_ref[lo:hi, 5, :]

        def step(s, keep):
            m = jnp.max(s, axis=1, keepdims=True)
            idx = jnp.argmax(s, axis=1, keepdims=True)
            one = iota == idx
            valid = jnp.where(m > 0.0, 1.0, 0.0)
            bx1 = jnp.sum(jnp.where(one, x1, 0.0), axis=1, keepdims=True)
            by1 = jnp.sum(jnp.where(one, y1, 0.0), axis=1, keepdims=True)
            bx2 = jnp.sum(jnp.where(one, x2, 0.0), axis=1, keepdims=True)
            by2 = jnp.sum(jnp.where(one, y2, 0.0), axis=1, keepdims=True)
            barea = jnp.maximum(bx2 - bx1, 0.0) * \
                jnp.maximum(by2 - by1, 0.0)
            ix1 = jnp.maximum(bx1, x1)
            iy1 = jnp.maximum(by1, y1)
            ix2 = jnp.minimum(bx2, x2)
            iy2 = jnp.minimum(by2, y2)
            inter = jnp.maximum(ix2 - ix1, 0.0) * \
                jnp.maximum(iy2 - iy1, 0.0)
            supp = 3.0 * inter > area + (barea + 1e-9)
            s = jnp.where(one | supp, 0.0, s)
            keep = jnp.where(one, valid, keep)
            return s, keep

        return s0, step

    s0a, step_a = make_half(0, _H)
    s0b, step_b = make_half(_H, _B)
    zeros = jnp.zeros((_H, _N), jnp.float32)

    def body(_, carry):
        sa, ka, sb, kb = carry
        sa, ka = step_a(sa, ka)
        sb, kb = step_b(sb, kb)
        return sa, ka, sb, kb

    _, ka, _, kb = jax.lax.fori_loop(
        0, _MAX_DET, body, (s0a, zeros, s0b, zeros), unroll=16)
    fs_ref[0:_H, :] = s0a * ka
    fs_ref[_H:_B, :] = s0b * kb


def _fused_kernel(b_ref, w_ref, feats_ref, prop_ref, out_ref, fs_ref,
                  stash_ref):
    b = pl.program_id(0)
    feats = feats_ref[0]
    ot = jax.lax.dot_general(
        w_ref[...], feats, (((1,), (1,)), ((), ())),
        preferred_element_type=jnp.float32)

    l0 = ot[0:1, :] + b_ref[0]
    l1 = ot[1:2, :] + b_ref[1]
    mx = jnp.maximum(l0, l1)
    e0 = jnp.exp(l0 - mx)
    e1 = jnp.exp(l1 - mx)
    score = e1 / (e0 + e1)
    score = jnp.where(score > _THRESH, score, 0.0)

    x1 = prop_ref[0, 0:1, :]
    y1 = prop_ref[0, 1:2, :]
    x2 = prop_ref[0, 2:3, :]
    y2 = prop_ref[0, 3:4, :]
    wd = x2 - x1
    ht = y2 - y1
    cx = x1 + 0.5 * wd
    cy = y1 + 0.5 * ht
    dx = (ot[2:3, :] + b_ref[2]) / 10.0
    dy = (ot[3:4, :] + b_ref[3]) / 10.0
    dw = jnp.minimum((ot[4:5, :] + b_ref[4]) / 5.0, _CLIP)
    dh = jnp.minimum((ot[5:6, :] + b_ref[5]) / 5.0, _CLIP)
    pcx = dx * wd + cx
    pcy = dy * ht + cy
    pw = jnp.exp(dw) * wd
    ph = jnp.exp(dh) * ht
    bx1 = jnp.clip(pcx - 0.5 * pw, 0.0, _IMG)
    by1 = jnp.clip(pcy - 0.5 * ph, 0.0, _IMG)
    bx2 = jnp.clip(pcx + 0.5 * pw, 0.0, _IMG)
    by2 = jnp.clip(pcy + 0.5 * ph, 0.0, _IMG)
    area = jnp.maximum(bx2 - bx1, 0.0) * jnp.maximum(by2 - by1, 0.0)

    val = jnp.concatenate(
        [bx1, by1, bx2, by2, score, area,
         jnp.zeros((2, _N), jnp.float32)], axis=0)
    out_ref[0] = val
    stash_ref[pl.ds(b, 1)] = val[None]

    @pl.when(b == _B - 1)
    def _():
        _run_nms(stash_ref, fs_ref)


def _forward(feats, proposals, W_cls, b_cls, W_box, b_box, interpret=False):
    w_all = jnp.concatenate([W_cls, W_box[4:8]], axis=0)
    w_pad = jnp.pad(w_all, ((0, 122), (0, 0)))
    b_all = jnp.concatenate(
        [b_cls, b_box[4:8], jnp.zeros((2,), jnp.float32)])
    prop_t = jnp.swapaxes(proposals, 1, 2)

    head, fs = pl.pallas_call(
        _fused_kernel,
        grid=(_B,),
        in_specs=[
            pl.BlockSpec(memory_space=pltpu.SMEM),
            pl.BlockSpec((128, _D), lambda i: (0, 0)),
            pl.BlockSpec((1, _N, _D), lambda i: (i, 0, 0)),
            pl.BlockSpec((1, 4, _N), lambda i: (i, 0, 0)),
        ],
        out_specs=[
            pl.BlockSpec((1, 8, _N), lambda i: (i, 0, 0)),
            pl.BlockSpec((_B, _N), lambda i: (0, 0)),
        ],
        out_shape=[
            jax.ShapeDtypeStruct((_B, 8, _N), jnp.float32),
            jax.ShapeDtypeStruct((_B, _N), jnp.float32),
        ],
        scratch_shapes=[
            pltpu.VMEM((_B, 8, _N), jnp.float32),
        ],
        compiler_params=pltpu.CompilerParams(
            dimension_semantics=("arbitrary",),
            vmem_limit_bytes=52 * 1024 * 1024,
        ),
        name="rcnn_fused",
        interpret=interpret,
    )(b_all, w_pad, feats, prop_t)

    out5 = jnp.concatenate([head[:, 0:4, :], fs[:, None, :]], axis=1)
    return jnp.swapaxes(out5, 1, 2)


def kernel(feats, proposals, W_cls, b_cls, W_box, b_box):
    return _forward(feats, proposals, W_cls, b_cls, W_box, b_box)
